# baseline (speedup 1.0000x reference)
import functools

import jax
import jax.numpy as jnp
from jax import lax
from jax.experimental import pallas as pl
from jax.experimental.pallas import tpu as pltpu
from jax.experimental.pallas import tpu_sc as plsc

N = 10000
NP = 10240
D = 128
E = 320000
NC = 2
NS = 16
NW = NC * NS
EPW = E // NW
EB = 40
NEB = EPW // EB
ROWS_PT = NP // NS
ZR = 128
DPT = NP // NS
NBUF = 5
NBI = 10
CHUNKS = NEB // NBI
EBD = 80
NEBD = EPW // EBD
UD = 5
CHUNKSD = NEBD // UD


@functools.cache
def _sc_mesh():
    return plsc.VectorSubcoreMesh(
        core_axis_name="c", subcore_axis_name="s",
        num_cores=NC, num_subcores=NS)


@functools.cache
def _sc_degrees_fn():
    return pl.kernel(
        _sc_degrees_body,
        out_type=[jax.ShapeDtypeStruct((NC * NP,), jnp.float32)
                  for _ in range(6)],
        mesh=_sc_mesh(),
        scratch_types=[
            [pltpu.VMEM((EBD,), jnp.int32) for _ in range(UD)],
            pltpu.VMEM((EBD,), jnp.float32),
            pltpu.VMEM((DPT,), jnp.float32),
            [pltpu.VMEM_SHARED((NP,), jnp.float32) for _ in range(6)],
            [pltpu.SemaphoreType.DMA for _ in range(UD)],
            [pltpu.SemaphoreType.DMA for _ in range(UD)],
        ],
    )


def _sc_degrees_body(s0, d0, s1, d1, s2, d2, o0, o1, o2, o3, o4, o5,
                     idxv, onesv, zv, hists, sli, ssc):
    c = lax.axis_index("c")
    s = lax.axis_index("s")
    wid = s * NC + c
    arrs = (s0, d0, s1, d1, s2, d2)
    outs = (o0, o1, o2, o3, o4, o5)

    def _init_ones(k, carry):
        onesv[pl.ds(k * 16, 16)] = jnp.full((16,), 1.0, jnp.float32)
        return carry
    lax.fori_loop(0, EBD // 16, _init_ones, 0)

    def _init_zero(k, carry):
        zv[pl.ds(k * 16, 16)] = jnp.zeros((16,), jnp.float32)
        return carry
    lax.fori_loop(0, DPT // 16, _init_zero, 0)

    for g in hists:
        pltpu.sync_copy(zv, g.at[pl.ds(s * DPT, DPT)])
    plsc.subcore_barrier()

    for arr, hist in zip(arrs, hists):
        for g in range(3):
            pltpu.async_copy(arr.at[wid, g], idxv[g], sli[g])

        def _chunk(ch, carry, arr=arr, hist=hist):
            for u in range(UD):
                blk = ch * UD + u
                u3 = (u + 3) % UD
                pltpu.make_async_copy(arr.at[wid, blk], idxv[u],
                                      sli[u]).wait()
                pltpu.async_copy(onesv, hist.at[idxv[u]], ssc[u], add=True)

                @pl.when(blk >= 2)
                def _wait_sc(hist=hist, u3=u3):
                    pltpu.make_async_copy(onesv, hist.at[idxv[u3]],
                                          ssc[u3]).wait()

                nxt = blk + 3

                @pl.when(nxt < NEBD)
                def _load_nxt(arr=arr, u3=u3, nxt=nxt):
                    pltpu.async_copy(arr.at[wid, nxt], idxv[u3], sli[u3])
            return carry
        lax.fori_loop(0, CHUNKSD, _chunk, 0)
        for blk in (NEBD - 2, NEBD - 1):
            pltpu.make_async_copy(onesv, hist.at[idxv[blk % UD]],
                                  ssc[blk % UD]).wait()
    plsc.subcore_barrier()

    for hist, o in zip(hists, outs):
        pltpu.sync_copy(hist.at[pl.ds(s * DPT, DPT)],
                        o.at[pl.ds(c * NP + s * DPT, DPT)])


@functools.cache
def _sc_aggregate_fn():
    return pl.kernel(
        _sc_aggregate_body,
        out_type=[jax.ShapeDtypeStruct((NC, NP, D), jnp.float32)
                  for _ in range(3)],
        mesh=_sc_mesh(),
        scratch_types=[
            [pltpu.VMEM((EB,), jnp.int32) for _ in range(NBI)],
            [pltpu.VMEM((EB,), jnp.int32) for _ in range(NBI)],
            [pltpu.VMEM((EB, D), jnp.float32) for _ in range(NBUF)],
            pltpu.VMEM((ZR, D), jnp.float32),
            pltpu.VMEM_SHARED((NP, D), jnp.float32),
            [pltpu.SemaphoreType.DMA for _ in range(NBI)],
            [pltpu.SemaphoreType.DMA for _ in range(NBI)],
            [pltpu.SemaphoreType.DMA for _ in range(NBUF)],
            [pltpu.SemaphoreType.DMA for _ in range(NBUF)],
        ],
    )


def _sc_aggregate_body(h0, h1, h2, s0, d0, s1, d1, s2, d2, o0, o1, o2,
                       srcv, dstv, rows, zrows, agg, sis, sid, sg, ss):
    c = lax.axis_index("c")
    s = lax.axis_index("s")
    wid = s * NC + c

    def _init_zero(k, carry):
        i = k // (D // 16)
        j = k % (D // 16)
        zrows[i, pl.ds(j * 16, 16)] = jnp.zeros((16,), jnp.float32)
        return carry
    lax.fori_loop(0, ZR * (D // 16), _init_zero, 0)

    for h, se, de, o in ((h0, s0, d0, o0), (h1, s1, d1, o1), (h2, s2, d2, o2)):
        for g in range(8):
            pltpu.async_copy(se.at[wid, g], srcv[g], sis[g])
            pltpu.async_copy(de.at[wid, g], dstv[g], sid[g])
        for g in range(4):
            pltpu.make_async_copy(se.at[wid, g], srcv[g], sis[g]).wait()
            pltpu.async_copy(h.at[srcv[g]], rows[g], sg[g])
        for k in range(ROWS_PT // ZR):
            pltpu.async_copy(zrows, agg.at[pl.ds(s * ROWS_PT + k * ZR, ZR)],
                             ss[k])
        for k in range(ROWS_PT // ZR):
            pltpu.make_async_copy(
                zrows, agg.at[pl.ds(s * ROWS_PT + k * ZR, ZR)], ss[k]).wait()
        plsc.subcore_barrier()

        def _chunk(ch, carry, h=h, se=se, de=de):
            for u in range(NBI):
                b = u % NBUF
                fb = (b + 4) % NBUF
                u4 = (u + 4) % NBI
                u8 = (u + 8) % NBI
                blk = ch * NBI + u
                pltpu.make_async_copy(h.at[srcv[u]], rows[b], sg[b]).wait()
                pltpu.make_async_copy(de.at[wid, blk], dstv[u],
                                      sid[u]).wait()
                pltpu.async_copy(rows[b], agg.at[dstv[u]], ss[b], add=True)

                @pl.when(blk >= 1)
                def _wait_prev(fb=fb, u=u):
                    pu = (u - 1) % NBI
                    pltpu.make_async_copy(rows[fb], agg.at[dstv[pu]],
                                          ss[fb]).wait()

                nxt4 = blk + 4

                @pl.when(nxt4 < NEB)
                def _next_gather(h=h, se=se, fb=fb, u4=u4, nxt4=nxt4):
                    pltpu.make_async_copy(se.at[wid, nxt4], srcv[u4],
                                          sis[u4]).wait()
                    pltpu.async_copy(h.at[srcv[u4]], rows[fb], sg[fb])

                nxt8 = blk + 8

                @pl.when(nxt8 < NEB)
                def _next_idx(se=se, de=de, u8=u8, nxt8=nxt8):
                    pltpu.async_copy(se.at[wid, nxt8], srcv[u8], sis[u8])
                    pltpu.async_copy(de.at[wid, nxt8], dstv[u8], sid[u8])
            return carry
        lax.fori_loop(0, CHUNKS, _chunk, 0)
        pltpu.make_async_copy(rows[(NEB - 1) % NBUF],
                              agg.at[dstv[(NEB - 1) % NBI]],
                              ss[(NEB - 1) % NBUF]).wait()
        plsc.subcore_barrier()

        pltpu.sync_copy(agg.at[pl.ds(s * ROWS_PT, ROWS_PT)],
                        o.at[c, pl.ds(s * ROWS_PT, ROWS_PT)])
        plsc.subcore_barrier()


BN = 2048
GRID = NP // BN


def _norm(dref):
    return lax.rsqrt(jnp.maximum(dref[0] + dref[1], 1.0))


def _tc0_body(x_ref, w_ref, o0, o1, o2):
    x = x_ref[...]
    for r, oref in enumerate((o0, o1, o2)):
        oref[...] = jnp.dot(x, w_ref[r], preferred_element_type=jnp.float32)


def _tc1b_body(p0, p1, p2, ds0, ds1, ds2, o0, o1, o2):
    for pref, dref, oref in ((p0, ds0, o0), (p1, ds1, o1), (p2, ds2, o2)):
        oref[...] = pref[...] * _norm(dref)


def _tc2_body(p0, p1, p2, di0, di1, di2, b_ref, ds0, ds1, ds2, w_ref,
              o0, o1, o2):
    h = jnp.zeros((BN, D), jnp.float32)
    for r, (pref, dref) in enumerate(((p0, di0), (p1, di1), (p2, di2))):
        h = h + (pref[0] + pref[1]) * _norm(dref) + b_ref[r][None, :]
    h = jnp.maximum(h, 0.0)
    for r, (dref, oref) in enumerate(((ds0, o0), (ds1, o1), (ds2, o2))):
        g = h * _norm(dref)
        oref[...] = jnp.dot(g, w_ref[r], preferred_element_type=jnp.float32)


def _tc3_body(q0, q1, q2, di0, di1, di2, b_ref, wl_ref, bl_ref, ohid, oout):
    h = jnp.zeros((BN, D), jnp.float32)
    for r, (qref, dref) in enumerate(((q0, di0), (q1, di1), (q2, di2))):
        h = h + (qref[0] + qref[1]) * _norm(dref) + b_ref[r][None, :]
    h = jnp.maximum(h, 0.0)
    ohid[...] = h
    oout[...] = (jnp.dot(h, wl_ref[...], preferred_element_type=jnp.float32)
                 + bl_ref[...])


_node_blk = pl.BlockSpec((BN, D), lambda i: (i, 0))
_deg_blk = pl.BlockSpec((2, BN, 1), lambda i: (0, i, 0))
_part_blk = pl.BlockSpec((2, BN, D), lambda i: (0, i, 0))
_w3_blk = pl.BlockSpec((3, D, D), lambda i: (0, 0, 0))
_b3_blk = pl.BlockSpec((3, D), lambda i: (0, 0))
_w_blk = pl.BlockSpec((D, D), lambda i: (0, 0))
_b_blk = pl.BlockSpec((1, D), lambda i: (0, 0))

_tc0 = pl.pallas_call(
    _tc0_body,
    grid=(GRID,),
    in_specs=[_node_blk, _w3_blk],
    out_specs=[_node_blk] * 3,
    out_shape=[jax.ShapeDtypeStruct((NP, D), jnp.float32)] * 3,
)

_tc1b = pl.pallas_call(
    _tc1b_body,
    grid=(GRID,),
    in_specs=[_node_blk] * 3 + [_deg_blk] * 3,
    out_specs=[_node_blk] * 3,
    out_shape=[jax.ShapeDtypeStruct((NP, D), jnp.float32)] * 3,
)

_tc2 = pl.pallas_call(
    _tc2_body,
    grid=(GRID,),
    in_specs=[_part_blk] * 3 + [_deg_blk] * 3 + [_b3_blk] + [_deg_blk] * 3
             + [_w3_blk],
    out_specs=[_node_blk] * 3,
    out_shape=[jax.ShapeDtypeStruct((NP, D), jnp.float32)] * 3,
)

_tc3 = pl.pallas_call(
    _tc3_body,
    grid=(GRID,),
    in_specs=[_part_blk] * 3 + [_deg_blk] * 3 + [_b3_blk, _w_blk, _b_blk],
    out_specs=[_node_blk] * 2,
    out_shape=[jax.ShapeDtypeStruct((NP, D), jnp.float32)] * 2,
)


def kernel(x, edge_index_r0, edge_index_r1, edge_index_r2,
           W1_0, b1_0, W1_1, b1_1, W1_2, b1_2,
           W2_0, b2_0, W2_1, b2_1, W2_2, b2_2, Wl, bl):
    s0, d0 = (a.reshape(NW, NEB, EB) for a in edge_index_r0)
    s1, d1 = (a.reshape(NW, NEB, EB) for a in edge_index_r1)
    s2, d2 = (a.reshape(NW, NEB, EB) for a in edge_index_r2)
    W1 = jnp.stack([W1_0, W1_1, W1_2])
    b1 = jnp.stack([b1_0, b1_1, b1_2])
    W2 = jnp.stack([W2_0, W2_1, W2_2])
    b2 = jnp.stack([b2_0, b2_1, b2_2])
    bl2 = bl.reshape(1, D)

    xp = jnp.pad(x, ((0, NP - N), (0, 0)))
    dg = [a.reshape(NW, NEBD, EBD)
          for e in (edge_index_r0, edge_index_r1, edge_index_r2) for a in e]
    degs = [g.reshape(NC, NP, 1)
            for g in _sc_degrees_fn()(*dg)]
    dsrc = [degs[0], degs[2], degs[4]]
    ddst = [degs[1], degs[3], degs[5]]

    P0, P1, P2 = _tc0(xp, W1)
    h0, h1, h2 = _tc1b(P0, P1, P2, dsrc[0], dsrc[1], dsrc[2])
    p0, p1, p2 = _sc_aggregate_fn()(h0, h1, h2, s0, d0, s1, d1, s2, d2)
    g0, g1, g2 = _tc2(p0, p1, p2, ddst[0], ddst[1], ddst[2], b1,
                      dsrc[0], dsrc[1], dsrc[2], W2)
    q0, q1, q2 = _sc_aggregate_fn()(g0, g1, g2, s0, d0, s1, d1, s2, d2)
    hidden, output = _tc3(q0, q1, q2, ddst[0], ddst[1], ddst[2], b2, Wl, bl2)
    return (hidden[:N], output[:N])


# speedup vs baseline: 1.0062x; 1.0062x over previous
import functools

import jax
import jax.numpy as jnp
from jax import lax
from jax.experimental import pallas as pl
from jax.experimental.pallas import tpu as pltpu
from jax.experimental.pallas import tpu_sc as plsc

N = 10000
NP = 10240
D = 128
E = 320000
NC = 2
NS = 16
NW = NC * NS
EPW = E // NW
EB = 40
NEB = EPW // EB
ROWS_PT = NP // NS
ZR = 128
DPT = NP // NS
NBUF = 5
NBI = 10
CHUNKS = NEB // NBI
EBD = 80
NEBD = EPW // EBD
UD = 5
CHUNKSD = NEBD // UD


@functools.cache
def _sc_mesh():
    return plsc.VectorSubcoreMesh(
        core_axis_name="c", subcore_axis_name="s",
        num_cores=NC, num_subcores=NS)


@functools.cache
def _sc_degrees_fn():
    return pl.kernel(
        _sc_degrees_body,
        out_type=[jax.ShapeDtypeStruct((NC * NP,), jnp.float32)
                  for _ in range(6)],
        mesh=_sc_mesh(),
        scratch_types=[
            [pltpu.VMEM((EBD,), jnp.int32) for _ in range(UD)],
            pltpu.VMEM((EBD,), jnp.float32),
            pltpu.VMEM((DPT,), jnp.float32),
            [pltpu.VMEM_SHARED((NP,), jnp.float32) for _ in range(6)],
            [pltpu.SemaphoreType.DMA for _ in range(UD)],
            [pltpu.SemaphoreType.DMA for _ in range(UD)],
        ],
    )


def _sc_degrees_body(s0, d0, s1, d1, s2, d2, o0, o1, o2, o3, o4, o5,
                     idxv, onesv, zv, hists, sli, ssc):
    c = lax.axis_index("c")
    s = lax.axis_index("s")
    wid = s * NC + c
    arrs = (s0, d0, s1, d1, s2, d2)
    outs = (o0, o1, o2, o3, o4, o5)

    def _init_ones(k, carry):
        onesv[pl.ds(k * 16, 16)] = jnp.full((16,), 1.0, jnp.float32)
        return carry
    lax.fori_loop(0, EBD // 16, _init_ones, 0)

    def _init_zero(k, carry):
        zv[pl.ds(k * 16, 16)] = jnp.zeros((16,), jnp.float32)
        return carry
    lax.fori_loop(0, DPT // 16, _init_zero, 0)

    for g in hists:
        pltpu.sync_copy(zv, g.at[pl.ds(s * DPT, DPT)])
    plsc.subcore_barrier()

    for arr, hist in zip(arrs, hists):
        for g in range(3):
            pltpu.async_copy(arr.at[wid, g], idxv[g], sli[g])

        def _chunk(ch, carry, arr=arr, hist=hist):
            for u in range(UD):
                blk = ch * UD + u
                u3 = (u + 3) % UD
                pltpu.make_async_copy(arr.at[wid, blk], idxv[u],
                                      sli[u]).wait()
                pltpu.async_copy(onesv, hist.at[idxv[u]], ssc[u], add=True)

                @pl.when(blk >= 2)
                def _wait_sc(hist=hist, u3=u3):
                    pltpu.make_async_copy(onesv, hist.at[idxv[u3]],
                                          ssc[u3]).wait()

                nxt = blk + 3

                @pl.when(nxt < NEBD)
                def _load_nxt(arr=arr, u3=u3, nxt=nxt):
                    pltpu.async_copy(arr.at[wid, nxt], idxv[u3], sli[u3])
            return carry
        lax.fori_loop(0, CHUNKSD, _chunk, 0)
        for blk in (NEBD - 2, NEBD - 1):
            pltpu.make_async_copy(onesv, hist.at[idxv[blk % UD]],
                                  ssc[blk % UD]).wait()
    plsc.subcore_barrier()

    for hist, o in zip(hists, outs):
        pltpu.sync_copy(hist.at[pl.ds(s * DPT, DPT)],
                        o.at[pl.ds(c * NP + s * DPT, DPT)])


@functools.cache
def _sc_aggregate_fn():
    return pl.kernel(
        _sc_aggregate_body,
        out_type=[jax.ShapeDtypeStruct((NC, NP, D), jnp.float32)
                  for _ in range(3)],
        mesh=_sc_mesh(),
        scratch_types=[
            [pltpu.VMEM((EB,), jnp.int32) for _ in range(NBI)],
            [pltpu.VMEM((EB,), jnp.int32) for _ in range(NBI)],
            [pltpu.VMEM((EB, D), jnp.float32) for _ in range(NBUF)],
            pltpu.VMEM((ZR, D), jnp.float32),
            pltpu.VMEM_SHARED((NP, D), jnp.float32),
            [pltpu.SemaphoreType.DMA for _ in range(NBI)],
            [pltpu.SemaphoreType.DMA for _ in range(NBI)],
            [pltpu.SemaphoreType.DMA for _ in range(NBUF)],
            [pltpu.SemaphoreType.DMA for _ in range(NBUF)],
        ],
    )


def _sc_aggregate_body(h0, h1, h2, s0, d0, s1, d1, s2, d2, o0, o1, o2,
                       srcv, dstv, rows, zrows, agg, sis, sid, sg, ss):
    c = lax.axis_index("c")
    s = lax.axis_index("s")
    wid = s * NC + c

    def _init_zero(k, carry):
        i = k // (D // 16)
        j = k % (D // 16)
        zrows[i, pl.ds(j * 16, 16)] = jnp.zeros((16,), jnp.float32)
        return carry
    lax.fori_loop(0, ZR * (D // 16), _init_zero, 0)

    for h, se, de, o in ((h0, s0, d0, o0), (h1, s1, d1, o1), (h2, s2, d2, o2)):
        for g in range(8):
            pltpu.async_copy(se.at[wid, g], srcv[g], sis[g])
            pltpu.async_copy(de.at[wid, g], dstv[g], sid[g])
        for g in range(4):
            pltpu.make_async_copy(se.at[wid, g], srcv[g], sis[g]).wait()
            pltpu.async_copy(h.at[srcv[g]], rows[g], sg[g])
        for k in range(ROWS_PT // ZR):
            pltpu.async_copy(zrows, agg.at[pl.ds(s * ROWS_PT + k * ZR, ZR)],
                             ss[k])
        for k in range(ROWS_PT // ZR):
            pltpu.make_async_copy(
                zrows, agg.at[pl.ds(s * ROWS_PT + k * ZR, ZR)], ss[k]).wait()
        plsc.subcore_barrier()

        def _chunk(ch, carry, h=h, se=se, de=de):
            for u in range(NBI):
                b = u % NBUF
                fb = (b + 4) % NBUF
                u4 = (u + 4) % NBI
                u8 = (u + 8) % NBI
                blk = ch * NBI + u
                pltpu.make_async_copy(h.at[srcv[u]], rows[b], sg[b]).wait()
                pltpu.make_async_copy(de.at[wid, blk], dstv[u],
                                      sid[u]).wait()
                pltpu.async_copy(rows[b], agg.at[dstv[u]], ss[b], add=True)

                @pl.when(blk >= 1)
                def _wait_prev(fb=fb, u=u):
                    pu = (u - 1) % NBI
                    pltpu.make_async_copy(rows[fb], agg.at[dstv[pu]],
                                          ss[fb]).wait()

                nxt4 = blk + 4

                @pl.when(nxt4 < NEB)
                def _next_gather(h=h, se=se, fb=fb, u4=u4, nxt4=nxt4):
                    pltpu.make_async_copy(se.at[wid, nxt4], srcv[u4],
                                          sis[u4]).wait()
                    pltpu.async_copy(h.at[srcv[u4]], rows[fb], sg[fb])

                nxt8 = blk + 8

                @pl.when(nxt8 < NEB)
                def _next_idx(se=se, de=de, u8=u8, nxt8=nxt8):
                    pltpu.async_copy(se.at[wid, nxt8], srcv[u8], sis[u8])
                    pltpu.async_copy(de.at[wid, nxt8], dstv[u8], sid[u8])
            return carry
        lax.fori_loop(0, CHUNKS, _chunk, 0)
        pltpu.make_async_copy(rows[(NEB - 1) % NBUF],
                              agg.at[dstv[(NEB - 1) % NBI]],
                              ss[(NEB - 1) % NBUF]).wait()
        plsc.subcore_barrier()

        pltpu.sync_copy(agg.at[pl.ds(s * ROWS_PT, ROWS_PT)],
                        o.at[c, pl.ds(s * ROWS_PT, ROWS_PT)])
        plsc.subcore_barrier()


BN = 2048
GRID = NP // BN


def _norm(dref):
    return lax.rsqrt(jnp.maximum(dref[0] + dref[1], 1.0))


def _tc1_body(x_ref, ds0, ds1, ds2, w_ref, o0, o1, o2):
    x = x_ref[...]
    for r, (dref, oref) in enumerate(((ds0, o0), (ds1, o1), (ds2, o2))):
        h = x * _norm(dref)
        oref[...] = jnp.dot(h, w_ref[r], preferred_element_type=jnp.float32)


def _tc2_body(p0, p1, p2, di0, di1, di2, b_ref, ds0, ds1, ds2, w_ref,
              o0, o1, o2):
    h = jnp.zeros((BN, D), jnp.float32)
    for r, (pref, dref) in enumerate(((p0, di0), (p1, di1), (p2, di2))):
        h = h + (pref[0] + pref[1]) * _norm(dref) + b_ref[r][None, :]
    h = jnp.maximum(h, 0.0)
    for r, (dref, oref) in enumerate(((ds0, o0), (ds1, o1), (ds2, o2))):
        g = h * _norm(dref)
        oref[...] = jnp.dot(g, w_ref[r], preferred_element_type=jnp.float32)


def _tc3_body(q0, q1, q2, di0, di1, di2, b_ref, wl_ref, bl_ref, ohid, oout):
    h = jnp.zeros((BN, D), jnp.float32)
    for r, (qref, dref) in enumerate(((q0, di0), (q1, di1), (q2, di2))):
        h = h + (qref[0] + qref[1]) * _norm(dref) + b_ref[r][None, :]
    h = jnp.maximum(h, 0.0)
    ohid[...] = h
    oout[...] = (jnp.dot(h, wl_ref[...], preferred_element_type=jnp.float32)
                 + bl_ref[...])


_node_blk = pl.BlockSpec((BN, D), lambda i: (i, 0))
_deg_blk = pl.BlockSpec((2, BN, 1), lambda i: (0, i, 0))
_part_blk = pl.BlockSpec((2, BN, D), lambda i: (0, i, 0))
_w3_blk = pl.BlockSpec((3, D, D), lambda i: (0, 0, 0))
_b3_blk = pl.BlockSpec((3, D), lambda i: (0, 0))
_w_blk = pl.BlockSpec((D, D), lambda i: (0, 0))
_b_blk = pl.BlockSpec((1, D), lambda i: (0, 0))

_tc1 = pl.pallas_call(
    _tc1_body,
    grid=(GRID,),
    in_specs=[_node_blk, _deg_blk, _deg_blk, _deg_blk, _w3_blk],
    out_specs=[_node_blk] * 3,
    out_shape=[jax.ShapeDtypeStruct((NP, D), jnp.float32)] * 3,
)

_tc2 = pl.pallas_call(
    _tc2_body,
    grid=(GRID,),
    in_specs=[_part_blk] * 3 + [_deg_blk] * 3 + [_b3_blk] + [_deg_blk] * 3
             + [_w3_blk],
    out_specs=[_node_blk] * 3,
    out_shape=[jax.ShapeDtypeStruct((NP, D), jnp.float32)] * 3,
)

_tc3 = pl.pallas_call(
    _tc3_body,
    grid=(GRID,),
    in_specs=[_part_blk] * 3 + [_deg_blk] * 3 + [_b3_blk, _w_blk, _b_blk],
    out_specs=[_node_blk] * 2,
    out_shape=[jax.ShapeDtypeStruct((NP, D), jnp.float32)] * 2,
)


def kernel(x, edge_index_r0, edge_index_r1, edge_index_r2,
           W1_0, b1_0, W1_1, b1_1, W1_2, b1_2,
           W2_0, b2_0, W2_1, b2_1, W2_2, b2_2, Wl, bl):
    s0, d0 = (a.reshape(NW, NEB, EB) for a in edge_index_r0)
    s1, d1 = (a.reshape(NW, NEB, EB) for a in edge_index_r1)
    s2, d2 = (a.reshape(NW, NEB, EB) for a in edge_index_r2)
    W1 = jnp.stack([W1_0, W1_1, W1_2])
    b1 = jnp.stack([b1_0, b1_1, b1_2])
    W2 = jnp.stack([W2_0, W2_1, W2_2])
    b2 = jnp.stack([b2_0, b2_1, b2_2])
    bl2 = bl.reshape(1, D)

    xp = jnp.pad(x, ((0, NP - N), (0, 0)))
    dg = [a.reshape(NW, NEBD, EBD)
          for e in (edge_index_r0, edge_index_r1, edge_index_r2) for a in e]
    degs = [g.reshape(NC, NP, 1)
            for g in _sc_degrees_fn()(*dg)]
    dsrc = [degs[0], degs[2], degs[4]]
    ddst = [degs[1], degs[3], degs[5]]

    h0, h1, h2 = _tc1(xp, dsrc[0], dsrc[1], dsrc[2], W1)
    p0, p1, p2 = _sc_aggregate_fn()(h0, h1, h2, s0, d0, s1, d1, s2, d2)
    g0, g1, g2 = _tc2(p0, p1, p2, ddst[0], ddst[1], ddst[2], b1,
                      dsrc[0], dsrc[1], dsrc[2], W2)
    q0, q1, q2 = _sc_aggregate_fn()(g0, g1, g2, s0, d0, s1, d1, s2, d2)
    hidden, output = _tc3(q0, q1, q2, ddst[0], ddst[1], ddst[2], b2, Wl, bl2)
    return (hidden[:N], output[:N])
